# baseline (speedup 1.0000x reference)
import functools

import numpy as np

import jax
import jax.numpy as jnp
from jax import lax
from jax.experimental import pallas as pl
from jax.experimental.pallas import tpu as pltpu
from jax.experimental.pallas import tpu_sc as plsc

N = 10000
V = 2230
H = 128
SEL = 2
D = H + SEL
ET = 6
PMAX = 4096
C = 2
G = 64
T = 2
E_CTRL, E_DATA, E_CALL = 80000, 64000, 16000
E = 2 * (E_CTRL + E_DATA + E_CALL)

DP = 144
NP = 10112
NC, NS = 2, 16
NW = NC * NS
CHUNK = 64
GG = 18
NG0 = 13
NG1 = 5
CPT0 = NG0 * GG
CPT1 = NG1 * GG
TOTC = NS * (CPT0 + CPT1)
E_PAD = TOTC * CHUNK
ROWS_PT = NP // NS
ECH = 128

RB = 1264
NBLK = NP // RB

_ET_NP = np.concatenate([
    np.full(E_CTRL, 0), np.full(E_DATA, 1), np.full(E_CALL, 2),
    np.full(E_CTRL, 3), np.full(E_DATA, 4), np.full(E_CALL, 5),
]).astype(np.int32)


def _make_edge_pass():
  mesh = plsc.VectorSubcoreMesh(core_axis_name="c", subcore_axis_name="s")

  def body(table, gidx, dstidx, init, out, gbuf, dbuf, rbuf, acc,
           gs0, gs1, gs2, ss0, ss1, ss2, isem):
    c = lax.axis_index("c")
    s = lax.axis_index("s")
    ng = jnp.where(c == 0, NG0, NG1)
    cbase = jnp.where(c == 0, s * CPT0, NS * CPT0 + s * CPT1)
    gsems = [gs0, gs1, gs2]
    ssems = [ss0, ss1, ss2]
    r0 = s * ROWS_PT
    pltpu.sync_copy(init.at[c, pl.ds(r0, ROWS_PT)], acc.at[pl.ds(r0, ROWS_PT)])
    pltpu.sync_copy(gidx.at[pl.ds(cbase, GG)], gbuf.at[0])
    pltpu.sync_copy(dstidx.at[pl.ds(cbase, GG)], dbuf.at[0])
    pltpu.async_copy(table.at[gbuf.at[0].at[0]], rbuf.at[0], gs0)
    pltpu.async_copy(table.at[gbuf.at[0].at[1]], rbuf.at[1], gs1)
    plsc.subcore_barrier()

    def group(g, _):
      gq = lax.rem(g, 2)
      gn = 1 - gq

      @pl.when(g < ng - 1)
      def _():
        pltpu.async_copy(gidx.at[pl.ds(cbase + (g + 1) * GG, GG)],
                         gbuf.at[gn], isem)
        pltpu.async_copy(dstidx.at[pl.ds(cbase + (g + 1) * GG, GG)],
                         dbuf.at[gn], isem)

      def step(jj, _):
        for q in range(3):
          o = jj * 3 + q
          cg = g * GG + o
          nq = (q + 2) % 3
          pltpu.make_async_copy(table.at[gbuf.at[gq].at[o]],
                                rbuf.at[q], gsems[q]).wait()
          pltpu.async_copy(rbuf.at[q], acc.at[dbuf.at[gq].at[o]],
                           ssems[q], add=True)
          @pl.when(jj * 3 + q + 2 < GG if q else cg >= 0)
          def _():
            @pl.when(cg >= 1)
            def _():
              pltpu.make_async_copy(rbuf.at[nq],
                                    acc.at[dbuf.at[gq].at[o]],
                                    ssems[nq]).wait()
            pltpu.async_copy(table.at[gbuf.at[gq].at[o + 2]],
                             rbuf.at[nq], gsems[nq])
        return 0

      lax.fori_loop(0, GG // 3, step, 0)

      @pl.when(g < ng - 1)
      def _():
        pltpu.make_async_copy(gidx.at[pl.ds(cbase + (g + 1) * GG, GG)],
                              gbuf.at[gn], isem).wait()
        pltpu.make_async_copy(dstidx.at[pl.ds(cbase + (g + 1) * GG, GG)],
                              dbuf.at[gn], isem).wait()
        pltpu.make_async_copy(rbuf.at[0], acc.at[dbuf.at[gn].at[0]],
                              ssems[0]).wait()
        pltpu.async_copy(table.at[gbuf.at[gn].at[0]], rbuf.at[0], gs0)
        pltpu.make_async_copy(rbuf.at[1], acc.at[dbuf.at[gn].at[1]],
                              ssems[1]).wait()
        pltpu.async_copy(table.at[gbuf.at[gn].at[1]], rbuf.at[1], gs1)
      return 0

    lax.fori_loop(0, ng, group, 0)
    for q in range(3):
      pltpu.make_async_copy(rbuf.at[q], acc.at[dbuf.at[0].at[0]],
                            ssems[q]).wait()
    plsc.subcore_barrier()
    pltpu.sync_copy(acc.at[pl.ds(r0, ROWS_PT)], out.at[c, pl.ds(r0, ROWS_PT)])

  return pl.kernel(
      body,
      out_type=jax.ShapeDtypeStruct((NC, NP, DP), jnp.float32),
      mesh=mesh,
      scratch_types=[
          pltpu.VMEM((2, GG, CHUNK), jnp.int32),
          pltpu.VMEM((2, GG, CHUNK), jnp.int32),
          pltpu.VMEM((3, CHUNK, DP), jnp.float32),
          pltpu.VMEM_SHARED((NP, DP), jnp.float32),
          pltpu.SemaphoreType.DMA,
          pltpu.SemaphoreType.DMA,
          pltpu.SemaphoreType.DMA,
          pltpu.SemaphoreType.DMA,
          pltpu.SemaphoreType.DMA,
          pltpu.SemaphoreType.DMA,
          pltpu.SemaphoreType.DMA,
      ],
      compiler_params=pltpu.CompilerParams(use_tc_tiling_on_sc=False),
  )


def _make_emb_gather():
  mesh = plsc.VectorSubcoreMesh(core_axis_name="c", subcore_axis_name="s")
  nchunk = NP // ECH

  def body(emb, vid, out, ibuf, rbuf, sem):
    c = lax.axis_index("c")
    s = lax.axis_index("s")
    wid = s * NC + c

    def step(k, _):
      j = wid + k * NW

      @pl.when(j < nchunk)
      def _():
        pltpu.sync_copy(vid.at[j], ibuf)
        pltpu.async_copy(emb.at[ibuf.at[0]], rbuf, sem).wait()
        pltpu.sync_copy(rbuf, out.at[pl.ds(j * ECH, ECH)])
      return 0

    lax.fori_loop(0, (nchunk + NW - 1) // NW, step, 0)

  return pl.kernel(
      body,
      out_type=jax.ShapeDtypeStruct((NP, H), jnp.float32),
      mesh=mesh,
      scratch_types=[
          pltpu.VMEM((1, ECH), jnp.int32),
          pltpu.VMEM((ECH, H), jnp.float32),
          pltpu.SemaphoreType.DMA,
      ],
      compiler_params=pltpu.CompilerParams(use_tc_tiling_on_sc=False),
  )


def _h0_body(embp_ref, selid_ref, selp_ref, out_ref):
  mask = selid_ref[...] == 0
  row = jnp.where(mask, selp_ref[0:1, :], selp_ref[1:2, :])
  out_ref[...] = embp_ref[...] + row


def _h0_assemble(emb_pad, selid, selp):
  return pl.pallas_call(
      _h0_body,
      grid=(NBLK,),
      in_specs=[
          pl.BlockSpec((RB, DP), lambda i: (i, 0)),
          pl.BlockSpec((RB, 1), lambda i: (i, 0)),
          pl.BlockSpec((2, DP), lambda i: (0, 0)),
      ],
      out_specs=pl.BlockSpec((RB, DP), lambda i: (i, 0)),
      out_shape=jax.ShapeDtypeStruct((NP, DP), jnp.float32),
  )(emb_pad, selid, selp)


def _postab_body(pos_ref, b_ref, out_ref):
  t = pl.program_id(0)
  out_ref[...] = pos_ref[...] + b_ref[pl.ds(t, 1), :]


def _postab_build(pos_pad, b_pad):
  blk = 1024
  return pl.pallas_call(
      _postab_body,
      grid=(ET, PMAX // blk),
      in_specs=[
          pl.BlockSpec((blk, DP), lambda t, p: (p, 0)),
          pl.BlockSpec((ET, DP), lambda t, p: (0, 0)),
      ],
      out_specs=pl.BlockSpec((blk, DP), lambda t, p: (t * (PMAX // blk) + p, 0)),
      out_shape=jax.ShapeDtypeStruct((ET * PMAX, DP), jnp.float32),
  )(pos_pad, b_pad)


def _trans_body(h_ref, w_ref, out_ref):
  out_ref[...] = jnp.dot(h_ref[...], w_ref[0],
                         preferred_element_type=jnp.float32)


def _trans(h, w_pad):
  return pl.pallas_call(
      _trans_body,
      grid=(ET, NBLK),
      in_specs=[
          pl.BlockSpec((RB, DP), lambda t, i: (i, 0)),
          pl.BlockSpec((1, DP, DP), lambda t, i: (t, 0, 0)),
      ],
      out_specs=pl.BlockSpec((RB, DP), lambda t, i: (t * NBLK + i, 0)),
      out_shape=jax.ShapeDtypeStruct((ET * NP, DP), jnp.float32),
  )(h, w_pad)


def _gru_body(ap_ref, h_ref, wir_ref, wiz_ref, win_ref, whr_ref, whz_ref,
              whn_ref, bir_ref, biz_ref, bin_ref, bhr_ref, bhz_ref, bhn_ref,
              out_ref):
  agg = ap_ref[0] + ap_ref[1]
  h = h_ref[...]
  dot = lambda a, b: jnp.dot(a, b, preferred_element_type=jnp.float32)
  r = jax.nn.sigmoid(dot(agg, wir_ref[...]) + bir_ref[...]
                     + dot(h, whr_ref[...]) + bhr_ref[...])
  z = jax.nn.sigmoid(dot(agg, wiz_ref[...]) + biz_ref[...]
                     + dot(h, whz_ref[...]) + bhz_ref[...])
  n = jnp.tanh(dot(agg, win_ref[...]) + bin_ref[...]
               + r * (dot(h, whn_ref[...]) + bhn_ref[...]))
  out_ref[...] = (1.0 - z) * n + z * h


def _gru(apart, h, wi3, wh3, bi3, bh3):
  mat = lambda: pl.BlockSpec((DP, DP), lambda i: (0, 0))
  vec = lambda: pl.BlockSpec((1, DP), lambda i: (0, 0))
  return pl.pallas_call(
      _gru_body,
      grid=(NBLK,),
      in_specs=[
          pl.BlockSpec((2, RB, DP), lambda i: (0, i, 0)),
          pl.BlockSpec((RB, DP), lambda i: (i, 0)),
          mat(), mat(), mat(), mat(), mat(), mat(),
          vec(), vec(), vec(), vec(), vec(), vec(),
      ],
      out_specs=pl.BlockSpec((RB, DP), lambda i: (i, 0)),
      out_shape=jax.ShapeDtypeStruct((NP, DP), jnp.float32),
  )(apart, h, wi3[0], wi3[1], wi3[2], wh3[0], wh3[1], wh3[2],
    bi3[0], bi3[1], bi3[2], bh3[0], bh3[1], bh3[2])


def _readout_body(h_ref, h0_ref, gb_ref, wi1_ref, wi2_ref, wj_ref, bi_ref,
                  bj_ref, out_ref):
  i = pl.program_id(0)
  dot = lambda a, b: jnp.dot(a, b, preferred_element_type=jnp.float32)
  gate = jax.nn.sigmoid(dot(h_ref[...], wi1_ref[...])
                        + dot(h0_ref[...], wi2_ref[...]) + bi_ref[...])
  val = dot(h_ref[...], wj_ref[...]) + bj_ref[...]
  gv = gate * val
  gidx = jax.lax.broadcasted_iota(jnp.int32, (RB, G), 1)
  oh = (gb_ref[...] == gidx).astype(jnp.float32)
  part = lax.dot_general(oh, gv, (((0,), (0,)), ((), ())),
                         preferred_element_type=jnp.float32)

  @pl.when(i == 0)
  def _():
    out_ref[...] = part

  @pl.when(i > 0)
  def _():
    out_ref[...] = out_ref[...] + part


def _readout(h, h0, gb, wi1, wi2, wj, bi, bj):
  return pl.pallas_call(
      _readout_body,
      grid=(NBLK,),
      in_specs=[
          pl.BlockSpec((RB, DP), lambda i: (i, 0)),
          pl.BlockSpec((RB, DP), lambda i: (i, 0)),
          pl.BlockSpec((RB, 1), lambda i: (i, 0)),
          pl.BlockSpec((DP, 128), lambda i: (0, 0)),
          pl.BlockSpec((DP, 128), lambda i: (0, 0)),
          pl.BlockSpec((DP, 128), lambda i: (0, 0)),
          pl.BlockSpec((1, 128), lambda i: (0, 0)),
          pl.BlockSpec((1, 128), lambda i: (0, 0)),
      ],
      out_specs=pl.BlockSpec((G, 128), lambda i: (0, 0)),
      out_shape=jax.ShapeDtypeStruct((G, 128), jnp.float32),
  )(h, h0, gb, wi1, wi2, wj, bi, bj)


def kernel(node_vocab_ids, node_selector_ids, control_edge_index,
           data_edge_index, call_edge_index, control_pos, data_pos,
           call_pos, graph_batch, params):
  f32 = jnp.float32
  i32 = jnp.int32

  src_f = jnp.concatenate([control_edge_index[0], data_edge_index[0],
                           call_edge_index[0]])
  dst_f = jnp.concatenate([control_edge_index[1], data_edge_index[1],
                           call_edge_index[1]])
  pos_f = jnp.concatenate([control_pos, data_pos, call_pos])
  src = jnp.concatenate([src_f, dst_f])
  dst = jnp.concatenate([dst_f, src_f])
  posa = jnp.concatenate([pos_f, pos_f])
  et = jnp.asarray(_ET_NP)

  gidx = src + et * NP
  pidx = posa + et * PMAX
  npad = E_PAD - E
  gidx2 = jnp.concatenate([gidx, jnp.zeros((npad,), i32)]).reshape(
      TOTC, CHUNK)
  pidx2 = jnp.concatenate([pidx, jnp.zeros((npad,), i32)]).reshape(
      TOTC, CHUNK)
  dst2 = jnp.concatenate([dst, jnp.full((npad,), N, i32)]).reshape(
      TOTC, CHUNK)

  vid = jnp.concatenate([node_vocab_ids,
                         jnp.zeros((NP - N,), i32)]).reshape(-1, 1, ECH)
  selid = jnp.concatenate([node_selector_ids,
                           jnp.zeros((NP - N,), i32)]).reshape(NP, 1)
  gb = jnp.concatenate([graph_batch,
                        jnp.full((NP - N,), G, i32)]).reshape(NP, 1)

  selp = jnp.zeros((2, DP), f32).at[:, H:H + SEL].set(params['sel'])
  pos_pad = jnp.pad(params['pos'], ((0, 0), (0, DP - D)))
  b_pad = jnp.pad(params['b'], ((0, 0), (0, DP - D)))
  w_pad = jnp.pad(params['W'], ((0, 0), (0, DP - D), (0, DP - D)))
  wih = params['Wih']
  whh = params['Whh']
  bih = params['bih']
  bhh = params['bhh']
  padm = lambda m: jnp.pad(m, ((0, DP - D), (0, DP - D)))
  padv = lambda v: jnp.pad(v, (0, DP - D)).reshape(1, DP)
  wi3 = [padm(wih[:, k * D:(k + 1) * D]) for k in range(3)]
  wh3 = [padm(whh[:, k * D:(k + 1) * D]) for k in range(3)]
  bi3 = [padv(bih[k * D:(k + 1) * D]) for k in range(3)]
  bh3 = [padv(bhh[k * D:(k + 1) * D]) for k in range(3)]
  wi1 = jnp.zeros((DP, 128), f32).at[:D, :C].set(params['Wi'][:D])
  wi2 = jnp.zeros((DP, 128), f32).at[:D, :C].set(params['Wi'][D:])
  wj = jnp.zeros((DP, 128), f32).at[:D, :C].set(params['Wj'])
  bi = jnp.zeros((1, 128), f32).at[0, :C].set(params['bi'])
  bj = jnp.zeros((1, 128), f32).at[0, :C].set(params['bj'])

  edge_pass = _make_edge_pass()
  emb_rows = _make_emb_gather()(params['emb'], vid)
  emb_pad = jnp.pad(emb_rows, ((0, 0), (0, DP - H)))
  h0 = _h0_assemble(emb_pad, selid, selp)
  postab = _postab_build(pos_pad, b_pad)

  zinit = jnp.zeros((NC, NP, DP), f32)
  ppart = edge_pass(postab, pidx2, dst2, zinit)

  h = h0
  for _ in range(T):
    tr = _trans(h, w_pad)
    apart = edge_pass(tr, gidx2, dst2, ppart)
    h = _gru(apart, h, wi3, wh3, bi3, bh3)

  logits = _readout(h, h0, gb, wi1, wi2, wj, bi, bj)
  return logits[:, :C]


# speedup vs baseline: 3.3852x; 3.3852x over previous
import functools

import numpy as np

import jax
import jax.numpy as jnp
from jax import lax
from jax.experimental import pallas as pl
from jax.experimental.pallas import tpu as pltpu
from jax.experimental.pallas import tpu_sc as plsc

N = 10000
V = 2230
H = 128
SEL = 2
D = H + SEL
ET = 6
PMAX = 4096
C = 2
G = 64
T = 2
E_CTRL, E_DATA, E_CALL = 80000, 64000, 16000
E = 2 * (E_CTRL + E_DATA + E_CALL)

DP = 144
NP = 10112
NC, NS = 2, 16
NW = NC * NS
CHUNK = 64
GG = 18
NG0 = 9
NG1 = 9
CPT0 = NG0 * GG
CPT1 = NG1 * GG
TOTC = NS * (CPT0 + CPT1)
E_PAD = TOTC * CHUNK
ROWS_PT = NP // NS
ECH = 128

RB = 1264
NBLK = NP // RB

_ET_NP = np.concatenate([
    np.full(E_CTRL, 0), np.full(E_DATA, 1), np.full(E_CALL, 2),
    np.full(E_CTRL, 3), np.full(E_DATA, 4), np.full(E_CALL, 5),
]).astype(np.int32)


def _make_edge_pass():
  mesh = plsc.VectorSubcoreMesh(core_axis_name="c", subcore_axis_name="s")

  def body(table, gidx, dstidx, init, out, gbuf, dbuf, rbuf, acc,
           gs0, gs1, gs2, ss0, ss1, ss2, isem):
    c = lax.axis_index("c")
    s = lax.axis_index("s")
    ng = jnp.where(c == 0, NG0, NG1)
    cbase = jnp.where(c == 0, s * CPT0, NS * CPT0 + s * CPT1)
    gsems = [gs0, gs1, gs2]
    ssems = [ss0, ss1, ss2]
    r0 = s * ROWS_PT
    pltpu.sync_copy(init.at[c, pl.ds(r0, ROWS_PT)], acc.at[pl.ds(r0, ROWS_PT)])
    pltpu.sync_copy(gidx.at[pl.ds(cbase, GG)], gbuf.at[0])
    pltpu.sync_copy(dstidx.at[pl.ds(cbase, GG)], dbuf.at[0])
    pltpu.async_copy(table.at[gbuf.at[0].at[0]], rbuf.at[0], gs0)
    pltpu.async_copy(table.at[gbuf.at[0].at[1]], rbuf.at[1], gs1)
    plsc.subcore_barrier()

    def group(g, _):
      gq = lax.rem(g, 2)
      gn = 1 - gq

      @pl.when(g < ng - 1)
      def _():
        pltpu.async_copy(gidx.at[pl.ds(cbase + (g + 1) * GG, GG)],
                         gbuf.at[gn], isem)
        pltpu.async_copy(dstidx.at[pl.ds(cbase + (g + 1) * GG, GG)],
                         dbuf.at[gn], isem)

      def step(jj, _):
        for q in range(3):
          o = jj * 3 + q
          cg = g * GG + o
          nq = (q + 2) % 3
          pltpu.make_async_copy(table.at[gbuf.at[gq].at[o]],
                                rbuf.at[q], gsems[q]).wait()
          pltpu.async_copy(rbuf.at[q], acc.at[dbuf.at[gq].at[o]],
                           ssems[q], add=True)
          @pl.when(jj * 3 + q + 2 < GG if q else cg >= 0)
          def _():
            @pl.when(cg >= 1)
            def _():
              pltpu.make_async_copy(rbuf.at[nq],
                                    acc.at[dbuf.at[gq].at[o]],
                                    ssems[nq]).wait()
            pltpu.async_copy(table.at[gbuf.at[gq].at[o + 2]],
                             rbuf.at[nq], gsems[nq])
        return 0

      lax.fori_loop(0, GG // 3, step, 0)

      @pl.when(g < ng - 1)
      def _():
        pltpu.make_async_copy(gidx.at[pl.ds(cbase + (g + 1) * GG, GG)],
                              gbuf.at[gn], isem).wait()
        pltpu.make_async_copy(dstidx.at[pl.ds(cbase + (g + 1) * GG, GG)],
                              dbuf.at[gn], isem).wait()
        pltpu.make_async_copy(rbuf.at[0], acc.at[dbuf.at[gn].at[0]],
                              ssems[0]).wait()
        pltpu.async_copy(table.at[gbuf.at[gn].at[0]], rbuf.at[0], gs0)
        pltpu.make_async_copy(rbuf.at[1], acc.at[dbuf.at[gn].at[1]],
                              ssems[1]).wait()
        pltpu.async_copy(table.at[gbuf.at[gn].at[1]], rbuf.at[1], gs1)
      return 0

    lax.fori_loop(0, ng, group, 0)
    for q in range(3):
      pltpu.make_async_copy(rbuf.at[q], acc.at[dbuf.at[0].at[0]],
                            ssems[q]).wait()
    plsc.subcore_barrier()
    pltpu.sync_copy(acc.at[pl.ds(r0, ROWS_PT)], out.at[c, pl.ds(r0, ROWS_PT)])

  return pl.kernel(
      body,
      out_type=jax.ShapeDtypeStruct((NC, NP, DP), jnp.float32),
      mesh=mesh,
      scratch_types=[
          pltpu.VMEM((2, GG, CHUNK), jnp.int32),
          pltpu.VMEM((2, GG, CHUNK), jnp.int32),
          pltpu.VMEM((3, CHUNK, DP), jnp.float32),
          pltpu.VMEM_SHARED((NP, DP), jnp.float32),
          pltpu.SemaphoreType.DMA,
          pltpu.SemaphoreType.DMA,
          pltpu.SemaphoreType.DMA,
          pltpu.SemaphoreType.DMA,
          pltpu.SemaphoreType.DMA,
          pltpu.SemaphoreType.DMA,
          pltpu.SemaphoreType.DMA,
      ],
      compiler_params=pltpu.CompilerParams(use_tc_tiling_on_sc=False),
  )


def _make_emb_gather():
  mesh = plsc.VectorSubcoreMesh(core_axis_name="c", subcore_axis_name="s")
  nchunk = NP // ECH

  def body(emb, vid, out, ibuf, rbuf, sem):
    c = lax.axis_index("c")
    s = lax.axis_index("s")
    wid = s * NC + c

    def step(k, _):
      j = wid + k * NW

      @pl.when(j < nchunk)
      def _():
        pltpu.sync_copy(vid.at[j], ibuf)
        pltpu.async_copy(emb.at[ibuf.at[0]], rbuf, sem).wait()
        pltpu.sync_copy(rbuf, out.at[pl.ds(j * ECH, ECH)])
      return 0

    lax.fori_loop(0, (nchunk + NW - 1) // NW, step, 0)

  return pl.kernel(
      body,
      out_type=jax.ShapeDtypeStruct((NP, H), jnp.float32),
      mesh=mesh,
      scratch_types=[
          pltpu.VMEM((1, ECH), jnp.int32),
          pltpu.VMEM((ECH, H), jnp.float32),
          pltpu.SemaphoreType.DMA,
      ],
      compiler_params=pltpu.CompilerParams(use_tc_tiling_on_sc=False),
  )


def _h0_body(embp_ref, selid_ref, selp_ref, out_ref):
  mask = selid_ref[...] == 0
  row = jnp.where(mask, selp_ref[0:1, :], selp_ref[1:2, :])
  out_ref[...] = embp_ref[...] + row


def _h0_assemble(emb_pad, selid, selp):
  return pl.pallas_call(
      _h0_body,
      grid=(NBLK,),
      in_specs=[
          pl.BlockSpec((RB, DP), lambda i: (i, 0)),
          pl.BlockSpec((RB, 1), lambda i: (i, 0)),
          pl.BlockSpec((2, DP), lambda i: (0, 0)),
      ],
      out_specs=pl.BlockSpec((RB, DP), lambda i: (i, 0)),
      out_shape=jax.ShapeDtypeStruct((NP, DP), jnp.float32),
  )(emb_pad, selid, selp)


def _postab_body(pos_ref, b_ref, out_ref):
  t = pl.program_id(0)
  out_ref[...] = pos_ref[...] + b_ref[pl.ds(t, 1), :]


def _postab_build(pos_pad, b_pad):
  blk = 1024
  return pl.pallas_call(
      _postab_body,
      grid=(ET, PMAX // blk),
      in_specs=[
          pl.BlockSpec((blk, DP), lambda t, p: (p, 0)),
          pl.BlockSpec((ET, DP), lambda t, p: (0, 0)),
      ],
      out_specs=pl.BlockSpec((blk, DP), lambda t, p: (t * (PMAX // blk) + p, 0)),
      out_shape=jax.ShapeDtypeStruct((ET * PMAX, DP), jnp.float32),
  )(pos_pad, b_pad)


def _trans_body(h_ref, w_ref, out_ref):
  out_ref[...] = jnp.dot(h_ref[...], w_ref[0],
                         preferred_element_type=jnp.float32)


def _trans(h, w_pad):
  return pl.pallas_call(
      _trans_body,
      grid=(ET, NBLK),
      in_specs=[
          pl.BlockSpec((RB, DP), lambda t, i: (i, 0)),
          pl.BlockSpec((1, DP, DP), lambda t, i: (t, 0, 0)),
      ],
      out_specs=pl.BlockSpec((RB, DP), lambda t, i: (t * NBLK + i, 0)),
      out_shape=jax.ShapeDtypeStruct((ET * NP, DP), jnp.float32),
  )(h, w_pad)


def _gru_body(ap_ref, h_ref, wir_ref, wiz_ref, win_ref, whr_ref, whz_ref,
              whn_ref, bir_ref, biz_ref, bin_ref, bhr_ref, bhz_ref, bhn_ref,
              out_ref):
  agg = ap_ref[0] + ap_ref[1]
  h = h_ref[...]
  dot = lambda a, b: jnp.dot(a, b, preferred_element_type=jnp.float32)
  r = jax.nn.sigmoid(dot(agg, wir_ref[...]) + bir_ref[...]
                     + dot(h, whr_ref[...]) + bhr_ref[...])
  z = jax.nn.sigmoid(dot(agg, wiz_ref[...]) + biz_ref[...]
                     + dot(h, whz_ref[...]) + bhz_ref[...])
  n = jnp.tanh(dot(agg, win_ref[...]) + bin_ref[...]
               + r * (dot(h, whn_ref[...]) + bhn_ref[...]))
  out_ref[...] = (1.0 - z) * n + z * h


def _gru(apart, h, wi3, wh3, bi3, bh3):
  mat = lambda: pl.BlockSpec((DP, DP), lambda i: (0, 0))
  vec = lambda: pl.BlockSpec((1, DP), lambda i: (0, 0))
  return pl.pallas_call(
      _gru_body,
      grid=(NBLK,),
      in_specs=[
          pl.BlockSpec((2, RB, DP), lambda i: (0, i, 0)),
          pl.BlockSpec((RB, DP), lambda i: (i, 0)),
          mat(), mat(), mat(), mat(), mat(), mat(),
          vec(), vec(), vec(), vec(), vec(), vec(),
      ],
      out_specs=pl.BlockSpec((RB, DP), lambda i: (i, 0)),
      out_shape=jax.ShapeDtypeStruct((NP, DP), jnp.float32),
  )(apart, h, wi3[0], wi3[1], wi3[2], wh3[0], wh3[1], wh3[2],
    bi3[0], bi3[1], bi3[2], bh3[0], bh3[1], bh3[2])


def _readout_body(h_ref, h0_ref, gb_ref, wi1_ref, wi2_ref, wj_ref, bi_ref,
                  bj_ref, out_ref):
  i = pl.program_id(0)
  dot = lambda a, b: jnp.dot(a, b, preferred_element_type=jnp.float32)
  gate = jax.nn.sigmoid(dot(h_ref[...], wi1_ref[...])
                        + dot(h0_ref[...], wi2_ref[...]) + bi_ref[...])
  val = dot(h_ref[...], wj_ref[...]) + bj_ref[...]
  gv = gate * val
  gidx = jax.lax.broadcasted_iota(jnp.int32, (RB, G), 1)
  oh = (gb_ref[...] == gidx).astype(jnp.float32)
  part = lax.dot_general(oh, gv, (((0,), (0,)), ((), ())),
                         preferred_element_type=jnp.float32)

  @pl.when(i == 0)
  def _():
    out_ref[...] = part

  @pl.when(i > 0)
  def _():
    out_ref[...] = out_ref[...] + part


def _readout(h, h0, gb, wi1, wi2, wj, bi, bj):
  return pl.pallas_call(
      _readout_body,
      grid=(NBLK,),
      in_specs=[
          pl.BlockSpec((RB, DP), lambda i: (i, 0)),
          pl.BlockSpec((RB, DP), lambda i: (i, 0)),
          pl.BlockSpec((RB, 1), lambda i: (i, 0)),
          pl.BlockSpec((DP, 128), lambda i: (0, 0)),
          pl.BlockSpec((DP, 128), lambda i: (0, 0)),
          pl.BlockSpec((DP, 128), lambda i: (0, 0)),
          pl.BlockSpec((1, 128), lambda i: (0, 0)),
          pl.BlockSpec((1, 128), lambda i: (0, 0)),
      ],
      out_specs=pl.BlockSpec((G, 128), lambda i: (0, 0)),
      out_shape=jax.ShapeDtypeStruct((G, 128), jnp.float32),
  )(h, h0, gb, wi1, wi2, wj, bi, bj)


def kernel(node_vocab_ids, node_selector_ids, control_edge_index,
           data_edge_index, call_edge_index, control_pos, data_pos,
           call_pos, graph_batch, params):
  f32 = jnp.float32
  i32 = jnp.int32

  src_f = jnp.concatenate([control_edge_index[0], data_edge_index[0],
                           call_edge_index[0]])
  dst_f = jnp.concatenate([control_edge_index[1], data_edge_index[1],
                           call_edge_index[1]])
  pos_f = jnp.concatenate([control_pos, data_pos, call_pos])
  src = jnp.concatenate([src_f, dst_f])
  dst = jnp.concatenate([dst_f, src_f])
  posa = jnp.concatenate([pos_f, pos_f])
  et = jnp.asarray(_ET_NP)

  gidx = src + et * NP
  pidx = posa + et * PMAX
  npad = E_PAD - E
  padr = np.arange(npad, dtype=np.int32)
  pad_gidx = jnp.asarray(padr % (ET * PMAX))
  pad_dst = jnp.asarray(N + padr % (NP - N))
  gidx2 = jnp.concatenate([gidx, pad_gidx]).reshape(TOTC, CHUNK)
  pidx2 = jnp.concatenate([pidx, pad_gidx]).reshape(TOTC, CHUNK)
  dst2 = jnp.concatenate([dst, pad_dst]).reshape(TOTC, CHUNK)

  vid = jnp.concatenate([node_vocab_ids,
                         jnp.zeros((NP - N,), i32)]).reshape(-1, 1, ECH)
  selid = jnp.concatenate([node_selector_ids,
                           jnp.zeros((NP - N,), i32)]).reshape(NP, 1)
  gb = jnp.concatenate([graph_batch,
                        jnp.full((NP - N,), G, i32)]).reshape(NP, 1)

  selp = jnp.zeros((2, DP), f32).at[:, H:H + SEL].set(params['sel'])
  pos_pad = jnp.pad(params['pos'], ((0, 0), (0, DP - D)))
  b_pad = jnp.pad(params['b'], ((0, 0), (0, DP - D)))
  w_pad = jnp.pad(params['W'], ((0, 0), (0, DP - D), (0, DP - D)))
  wih = params['Wih']
  whh = params['Whh']
  bih = params['bih']
  bhh = params['bhh']
  padm = lambda m: jnp.pad(m, ((0, DP - D), (0, DP - D)))
  padv = lambda v: jnp.pad(v, (0, DP - D)).reshape(1, DP)
  wi3 = [padm(wih[:, k * D:(k + 1) * D]) for k in range(3)]
  wh3 = [padm(whh[:, k * D:(k + 1) * D]) for k in range(3)]
  bi3 = [padv(bih[k * D:(k + 1) * D]) for k in range(3)]
  bh3 = [padv(bhh[k * D:(k + 1) * D]) for k in range(3)]
  wi1 = jnp.zeros((DP, 128), f32).at[:D, :C].set(params['Wi'][:D])
  wi2 = jnp.zeros((DP, 128), f32).at[:D, :C].set(params['Wi'][D:])
  wj = jnp.zeros((DP, 128), f32).at[:D, :C].set(params['Wj'])
  bi = jnp.zeros((1, 128), f32).at[0, :C].set(params['bi'])
  bj = jnp.zeros((1, 128), f32).at[0, :C].set(params['bj'])

  edge_pass = _make_edge_pass()
  emb_rows = _make_emb_gather()(params['emb'], vid)
  emb_pad = jnp.pad(emb_rows, ((0, 0), (0, DP - H)))
  h0 = _h0_assemble(emb_pad, selid, selp)
  postab = _postab_build(pos_pad, b_pad)

  zinit = jnp.zeros((NC, NP, DP), f32)
  ppart = edge_pass(postab, pidx2, dst2, zinit)

  h = h0
  for _ in range(T):
    tr = _trans(h, w_pad)
    apart = edge_pass(tr, gidx2, dst2, ppart)
    h = _gru(apart, h, wi3, wh3, bi3, bh3)

  logits = _readout(h, h0, gb, wi1, wi2, wj, bi, bj)
  return logits[:, :C]


# speedup vs baseline: 3.5724x; 1.0553x over previous
import functools

import numpy as np

import jax
import jax.numpy as jnp
from jax import lax
from jax.experimental import pallas as pl
from jax.experimental.pallas import tpu as pltpu
from jax.experimental.pallas import tpu_sc as plsc

N = 10000
V = 2230
H = 128
SEL = 2
D = H + SEL
ET = 6
PMAX = 4096
C = 2
G = 64
T = 2
E_CTRL, E_DATA, E_CALL = 80000, 64000, 16000
E = 2 * (E_CTRL + E_DATA + E_CALL)

DP = 144
NP = 10112
NC, NS = 2, 16
NW = NC * NS
CHUNK = 64
GG = 18
NG0 = 9
NG1 = 9
CPT0 = NG0 * GG
CPT1 = NG1 * GG
TOTC = NS * (CPT0 + CPT1)
E_PAD = TOTC * CHUNK
ROWS_PT = NP // NS
ECH = 128

RB = 1264
NBLK = NP // RB

_ET_NP = np.concatenate([
    np.full(E_CTRL, 0), np.full(E_DATA, 1), np.full(E_CALL, 2),
    np.full(E_CTRL, 3), np.full(E_DATA, 4), np.full(E_CALL, 5),
]).astype(np.int32)


def _make_edge_pass():
  mesh = plsc.VectorSubcoreMesh(core_axis_name="c", subcore_axis_name="s")

  def body(table, gidx, dstidx, init, out, gbuf, dbuf, rbuf, acc,
           gs0, gs1, gs2, ss0, ss1, ss2, isem):
    c = lax.axis_index("c")
    s = lax.axis_index("s")
    ng = jnp.where(c == 0, NG0, NG1)
    cbase = jnp.where(c == 0, s * CPT0, NS * CPT0 + s * CPT1)
    gsems = [gs0, gs1, gs2]
    ssems = [ss0, ss1, ss2]
    r0 = s * ROWS_PT
    pltpu.sync_copy(init.at[c, pl.ds(r0, ROWS_PT)], acc.at[pl.ds(r0, ROWS_PT)])
    pltpu.sync_copy(gidx.at[pl.ds(cbase, GG)], gbuf.at[0])
    pltpu.sync_copy(dstidx.at[pl.ds(cbase, GG)], dbuf.at[0])
    pltpu.async_copy(table.at[gbuf.at[0].at[0]], rbuf.at[0], gs0)
    pltpu.async_copy(table.at[gbuf.at[0].at[1]], rbuf.at[1], gs1)
    plsc.subcore_barrier()

    def group(g, _):
      gq = lax.rem(g, 2)
      gn = 1 - gq

      @pl.when(g < ng - 1)
      def _():
        pltpu.async_copy(gidx.at[pl.ds(cbase + (g + 1) * GG, GG)],
                         gbuf.at[gn], isem)
        pltpu.async_copy(dstidx.at[pl.ds(cbase + (g + 1) * GG, GG)],
                         dbuf.at[gn], isem)

      def step(jj, _):
        for q in range(3):
          o = jj * 3 + q
          cg = g * GG + o
          nq = (q + 2) % 3
          pltpu.make_async_copy(table.at[gbuf.at[gq].at[o]],
                                rbuf.at[q], gsems[q]).wait()
          pltpu.async_copy(rbuf.at[q], acc.at[dbuf.at[gq].at[o]],
                           ssems[q], add=True)
          @pl.when(jj * 3 + q + 2 < GG if q else cg >= 0)
          def _():
            @pl.when(cg >= 1)
            def _():
              pltpu.make_async_copy(rbuf.at[nq],
                                    acc.at[dbuf.at[gq].at[o]],
                                    ssems[nq]).wait()
            pltpu.async_copy(table.at[gbuf.at[gq].at[o + 2]],
                             rbuf.at[nq], gsems[nq])
        return 0

      lax.fori_loop(0, GG // 3, step, 0)

      @pl.when(g < ng - 1)
      def _():
        pltpu.make_async_copy(gidx.at[pl.ds(cbase + (g + 1) * GG, GG)],
                              gbuf.at[gn], isem).wait()
        pltpu.make_async_copy(dstidx.at[pl.ds(cbase + (g + 1) * GG, GG)],
                              dbuf.at[gn], isem).wait()
        pltpu.make_async_copy(rbuf.at[0], acc.at[dbuf.at[gn].at[0]],
                              ssems[0]).wait()
        pltpu.async_copy(table.at[gbuf.at[gn].at[0]], rbuf.at[0], gs0)
        pltpu.make_async_copy(rbuf.at[1], acc.at[dbuf.at[gn].at[1]],
                              ssems[1]).wait()
        pltpu.async_copy(table.at[gbuf.at[gn].at[1]], rbuf.at[1], gs1)
      return 0

    lax.fori_loop(0, ng, group, 0)
    for q in range(3):
      pltpu.make_async_copy(rbuf.at[q], acc.at[dbuf.at[0].at[0]],
                            ssems[q]).wait()
    plsc.subcore_barrier()
    pltpu.sync_copy(acc.at[pl.ds(r0, ROWS_PT)], out.at[c, pl.ds(r0, ROWS_PT)])

  return pl.kernel(
      body,
      out_type=jax.ShapeDtypeStruct((NC, NP, DP), jnp.float32),
      mesh=mesh,
      scratch_types=[
          pltpu.VMEM((2, GG, CHUNK), jnp.int32),
          pltpu.VMEM((2, GG, CHUNK), jnp.int32),
          pltpu.VMEM((3, CHUNK, DP), jnp.float32),
          pltpu.VMEM_SHARED((NP, DP), jnp.float32),
          pltpu.SemaphoreType.DMA,
          pltpu.SemaphoreType.DMA,
          pltpu.SemaphoreType.DMA,
          pltpu.SemaphoreType.DMA,
          pltpu.SemaphoreType.DMA,
          pltpu.SemaphoreType.DMA,
          pltpu.SemaphoreType.DMA,
      ],
      compiler_params=pltpu.CompilerParams(use_tc_tiling_on_sc=False),
  )


def _make_pos_pass():
  mesh = plsc.VectorSubcoreMesh(core_axis_name="c", subcore_axis_name="s")
  TROWS = PMAX // NS

  def body(table, gidx, dstidx, init, out, gbuf, dbuf, rbuf, acc, tab,
           gs0, gs1, gs2, ss0, ss1, ss2, isem):
    c = lax.axis_index("c")
    s = lax.axis_index("s")
    ng = jnp.where(c == 0, NG0, NG1)
    cbase = jnp.where(c == 0, s * CPT0, NS * CPT0 + s * CPT1)
    gsems = [gs0, gs1, gs2]
    ssems = [ss0, ss1, ss2]
    r0 = s * ROWS_PT
    pltpu.sync_copy(init.at[c, pl.ds(r0, ROWS_PT)], acc.at[pl.ds(r0, ROWS_PT)])
    t0 = s * TROWS
    pltpu.sync_copy(table.at[pl.ds(t0, TROWS)], tab.at[pl.ds(t0, TROWS)])
    pltpu.sync_copy(gidx.at[pl.ds(cbase, GG)], gbuf.at[0])
    pltpu.sync_copy(dstidx.at[pl.ds(cbase, GG)], dbuf.at[0])
    plsc.subcore_barrier()
    pltpu.async_copy(tab.at[gbuf.at[0].at[0]], rbuf.at[0], gs0)
    pltpu.async_copy(tab.at[gbuf.at[0].at[1]], rbuf.at[1], gs1)

    def group(g, _):
      gq = lax.rem(g, 2)
      gn = 1 - gq

      @pl.when(g < ng - 1)
      def _():
        pltpu.async_copy(gidx.at[pl.ds(cbase + (g + 1) * GG, GG)],
                         gbuf.at[gn], isem)
        pltpu.async_copy(dstidx.at[pl.ds(cbase + (g + 1) * GG, GG)],
                         dbuf.at[gn], isem)

      def step(jj, _):
        for q in range(3):
          o = jj * 3 + q
          cg = g * GG + o
          nq = (q + 2) % 3
          pltpu.make_async_copy(tab.at[gbuf.at[gq].at[o]],
                                rbuf.at[q], gsems[q]).wait()
          pltpu.async_copy(rbuf.at[q], acc.at[dbuf.at[gq].at[o]],
                           ssems[q], add=True)
          @pl.when(jj * 3 + q + 2 < GG if q else cg >= 0)
          def _():
            @pl.when(cg >= 1)
            def _():
              pltpu.make_async_copy(rbuf.at[nq],
                                    acc.at[dbuf.at[gq].at[o]],
                                    ssems[nq]).wait()
            pltpu.async_copy(tab.at[gbuf.at[gq].at[o + 2]],
                             rbuf.at[nq], gsems[nq])
        return 0

      lax.fori_loop(0, GG // 3, step, 0)

      @pl.when(g < ng - 1)
      def _():
        pltpu.make_async_copy(gidx.at[pl.ds(cbase + (g + 1) * GG, GG)],
                              gbuf.at[gn], isem).wait()
        pltpu.make_async_copy(dstidx.at[pl.ds(cbase + (g + 1) * GG, GG)],
                              dbuf.at[gn], isem).wait()
        pltpu.make_async_copy(rbuf.at[0], acc.at[dbuf.at[gn].at[0]],
                              ssems[0]).wait()
        pltpu.async_copy(tab.at[gbuf.at[gn].at[0]], rbuf.at[0], gs0)
        pltpu.make_async_copy(rbuf.at[1], acc.at[dbuf.at[gn].at[1]],
                              ssems[1]).wait()
        pltpu.async_copy(tab.at[gbuf.at[gn].at[1]], rbuf.at[1], gs1)
      return 0

    lax.fori_loop(0, ng, group, 0)
    for q in range(3):
      pltpu.make_async_copy(rbuf.at[q], acc.at[dbuf.at[0].at[0]],
                            ssems[q]).wait()
    plsc.subcore_barrier()
    pltpu.sync_copy(acc.at[pl.ds(r0, ROWS_PT)], out.at[c, pl.ds(r0, ROWS_PT)])

  return pl.kernel(
      body,
      out_type=jax.ShapeDtypeStruct((NC, NP, DP), jnp.float32),
      mesh=mesh,
      scratch_types=[
          pltpu.VMEM((2, GG, CHUNK), jnp.int32),
          pltpu.VMEM((2, GG, CHUNK), jnp.int32),
          pltpu.VMEM((3, CHUNK, DP), jnp.float32),
          pltpu.VMEM_SHARED((NP, DP), jnp.float32),
          pltpu.VMEM_SHARED((PMAX, DP), jnp.float32),
          pltpu.SemaphoreType.DMA,
          pltpu.SemaphoreType.DMA,
          pltpu.SemaphoreType.DMA,
          pltpu.SemaphoreType.DMA,
          pltpu.SemaphoreType.DMA,
          pltpu.SemaphoreType.DMA,
          pltpu.SemaphoreType.DMA,
      ],
      compiler_params=pltpu.CompilerParams(use_tc_tiling_on_sc=False),
  )


def _make_emb_gather():
  mesh = plsc.VectorSubcoreMesh(core_axis_name="c", subcore_axis_name="s")
  nchunk = NP // ECH

  def body(emb, vid, out, ibuf, rbuf, sem):
    c = lax.axis_index("c")
    s = lax.axis_index("s")
    wid = s * NC + c

    def step(k, _):
      j = wid + k * NW

      @pl.when(j < nchunk)
      def _():
        pltpu.sync_copy(vid.at[j], ibuf)
        pltpu.async_copy(emb.at[ibuf.at[0]], rbuf, sem).wait()
        pltpu.sync_copy(rbuf, out.at[pl.ds(j * ECH, ECH)])
      return 0

    lax.fori_loop(0, (nchunk + NW - 1) // NW, step, 0)

  return pl.kernel(
      body,
      out_type=jax.ShapeDtypeStruct((NP, H), jnp.float32),
      mesh=mesh,
      scratch_types=[
          pltpu.VMEM((1, ECH), jnp.int32),
          pltpu.VMEM((ECH, H), jnp.float32),
          pltpu.SemaphoreType.DMA,
      ],
      compiler_params=pltpu.CompilerParams(use_tc_tiling_on_sc=False),
  )


def _h0_body(embp_ref, selid_ref, selp_ref, out_ref):
  mask = selid_ref[...] == 0
  row = jnp.where(mask, selp_ref[0:1, :], selp_ref[1:2, :])
  out_ref[...] = embp_ref[...] + row


def _h0_assemble(emb_pad, selid, selp):
  return pl.pallas_call(
      _h0_body,
      grid=(NBLK,),
      in_specs=[
          pl.BlockSpec((RB, DP), lambda i: (i, 0)),
          pl.BlockSpec((RB, 1), lambda i: (i, 0)),
          pl.BlockSpec((2, DP), lambda i: (0, 0)),
      ],
      out_specs=pl.BlockSpec((RB, DP), lambda i: (i, 0)),
      out_shape=jax.ShapeDtypeStruct((NP, DP), jnp.float32),
  )(emb_pad, selid, selp)


def _trans_body(h_ref, w_ref, b_ref, out_ref):
  out_ref[...] = (jnp.dot(h_ref[...], w_ref[0],
                          preferred_element_type=jnp.float32)
                  + b_ref[0])


def _trans(h, w_pad, b_pad):
  b_pad = b_pad.reshape(ET, 1, DP)
  return pl.pallas_call(
      _trans_body,
      grid=(ET, NBLK),
      in_specs=[
          pl.BlockSpec((RB, DP), lambda t, i: (i, 0)),
          pl.BlockSpec((1, DP, DP), lambda t, i: (t, 0, 0)),
          pl.BlockSpec((1, 1, DP), lambda t, i: (t, 0, 0)),
      ],
      out_specs=pl.BlockSpec((RB, DP), lambda t, i: (t * NBLK + i, 0)),
      out_shape=jax.ShapeDtypeStruct((ET * NP, DP), jnp.float32),
  )(h, w_pad, b_pad)


def _gru_body(ap_ref, h_ref, wir_ref, wiz_ref, win_ref, whr_ref, whz_ref,
              whn_ref, bir_ref, biz_ref, bin_ref, bhr_ref, bhz_ref, bhn_ref,
              out_ref):
  agg = ap_ref[0] + ap_ref[1]
  h = h_ref[...]
  dot = lambda a, b: jnp.dot(a, b, preferred_element_type=jnp.float32)
  r = jax.nn.sigmoid(dot(agg, wir_ref[...]) + bir_ref[...]
                     + dot(h, whr_ref[...]) + bhr_ref[...])
  z = jax.nn.sigmoid(dot(agg, wiz_ref[...]) + biz_ref[...]
                     + dot(h, whz_ref[...]) + bhz_ref[...])
  n = jnp.tanh(dot(agg, win_ref[...]) + bin_ref[...]
               + r * (dot(h, whn_ref[...]) + bhn_ref[...]))
  out_ref[...] = (1.0 - z) * n + z * h


def _gru(apart, h, wi3, wh3, bi3, bh3):
  mat = lambda: pl.BlockSpec((DP, DP), lambda i: (0, 0))
  vec = lambda: pl.BlockSpec((1, DP), lambda i: (0, 0))
  return pl.pallas_call(
      _gru_body,
      grid=(NBLK,),
      in_specs=[
          pl.BlockSpec((2, RB, DP), lambda i: (0, i, 0)),
          pl.BlockSpec((RB, DP), lambda i: (i, 0)),
          mat(), mat(), mat(), mat(), mat(), mat(),
          vec(), vec(), vec(), vec(), vec(), vec(),
      ],
      out_specs=pl.BlockSpec((RB, DP), lambda i: (i, 0)),
      out_shape=jax.ShapeDtypeStruct((NP, DP), jnp.float32),
  )(apart, h, wi3[0], wi3[1], wi3[2], wh3[0], wh3[1], wh3[2],
    bi3[0], bi3[1], bi3[2], bh3[0], bh3[1], bh3[2])


def _readout_body(h_ref, h0_ref, gb_ref, wi1_ref, wi2_ref, wj_ref, bi_ref,
                  bj_ref, out_ref):
  i = pl.program_id(0)
  dot = lambda a, b: jnp.dot(a, b, preferred_element_type=jnp.float32)
  gate = jax.nn.sigmoid(dot(h_ref[...], wi1_ref[...])
                        + dot(h0_ref[...], wi2_ref[...]) + bi_ref[...])
  val = dot(h_ref[...], wj_ref[...]) + bj_ref[...]
  gv = gate * val
  gidx = jax.lax.broadcasted_iota(jnp.int32, (RB, G), 1)
  oh = (gb_ref[...] == gidx).astype(jnp.float32)
  part = lax.dot_general(oh, gv, (((0,), (0,)), ((), ())),
                         preferred_element_type=jnp.float32)

  @pl.when(i == 0)
  def _():
    out_ref[...] = part

  @pl.when(i > 0)
  def _():
    out_ref[...] = out_ref[...] + part


def _readout(h, h0, gb, wi1, wi2, wj, bi, bj):
  return pl.pallas_call(
      _readout_body,
      grid=(NBLK,),
      in_specs=[
          pl.BlockSpec((RB, DP), lambda i: (i, 0)),
          pl.BlockSpec((RB, DP), lambda i: (i, 0)),
          pl.BlockSpec((RB, 1), lambda i: (i, 0)),
          pl.BlockSpec((DP, 128), lambda i: (0, 0)),
          pl.BlockSpec((DP, 128), lambda i: (0, 0)),
          pl.BlockSpec((DP, 128), lambda i: (0, 0)),
          pl.BlockSpec((1, 128), lambda i: (0, 0)),
          pl.BlockSpec((1, 128), lambda i: (0, 0)),
      ],
      out_specs=pl.BlockSpec((G, 128), lambda i: (0, 0)),
      out_shape=jax.ShapeDtypeStruct((G, 128), jnp.float32),
  )(h, h0, gb, wi1, wi2, wj, bi, bj)


def kernel(node_vocab_ids, node_selector_ids, control_edge_index,
           data_edge_index, call_edge_index, control_pos, data_pos,
           call_pos, graph_batch, params):
  f32 = jnp.float32
  i32 = jnp.int32

  src_f = jnp.concatenate([control_edge_index[0], data_edge_index[0],
                           call_edge_index[0]])
  dst_f = jnp.concatenate([control_edge_index[1], data_edge_index[1],
                           call_edge_index[1]])
  pos_f = jnp.concatenate([control_pos, data_pos, call_pos])
  src = jnp.concatenate([src_f, dst_f])
  dst = jnp.concatenate([dst_f, src_f])
  posa = jnp.concatenate([pos_f, pos_f])
  et = jnp.asarray(_ET_NP)

  gidx = src + et * NP
  pidx = posa
  npad = E_PAD - E
  padr = np.arange(npad, dtype=np.int32)
  pad_gidx = jnp.asarray(padr % (ET * PMAX))
  pad_pidx = jnp.asarray(padr % PMAX)
  pad_dst = jnp.asarray(N + padr % (NP - N))
  gidx2 = jnp.concatenate([gidx, pad_gidx]).reshape(TOTC, CHUNK)
  pidx2 = jnp.concatenate([pidx, pad_pidx]).reshape(TOTC, CHUNK)
  dst2 = jnp.concatenate([dst, pad_dst]).reshape(TOTC, CHUNK)

  vid = jnp.concatenate([node_vocab_ids,
                         jnp.zeros((NP - N,), i32)]).reshape(-1, 1, ECH)
  selid = jnp.concatenate([node_selector_ids,
                           jnp.zeros((NP - N,), i32)]).reshape(NP, 1)
  gb = jnp.concatenate([graph_batch,
                        jnp.full((NP - N,), G, i32)]).reshape(NP, 1)

  selp = jnp.zeros((2, DP), f32).at[:, H:H + SEL].set(params['sel'])
  pos_pad = jnp.pad(params['pos'], ((0, 0), (0, DP - D)))
  b_pad = jnp.pad(params['b'], ((0, 0), (0, DP - D)))
  w_pad = jnp.pad(params['W'], ((0, 0), (0, DP - D), (0, DP - D)))
  wih = params['Wih']
  whh = params['Whh']
  bih = params['bih']
  bhh = params['bhh']
  padm = lambda m: jnp.pad(m, ((0, DP - D), (0, DP - D)))
  padv = lambda v: jnp.pad(v, (0, DP - D)).reshape(1, DP)
  wi3 = [padm(wih[:, k * D:(k + 1) * D]) for k in range(3)]
  wh3 = [padm(whh[:, k * D:(k + 1) * D]) for k in range(3)]
  bi3 = [padv(bih[k * D:(k + 1) * D]) for k in range(3)]
  bh3 = [padv(bhh[k * D:(k + 1) * D]) for k in range(3)]
  wi1 = jnp.zeros((DP, 128), f32).at[:D, :C].set(params['Wi'][:D])
  wi2 = jnp.zeros((DP, 128), f32).at[:D, :C].set(params['Wi'][D:])
  wj = jnp.zeros((DP, 128), f32).at[:D, :C].set(params['Wj'])
  bi = jnp.zeros((1, 128), f32).at[0, :C].set(params['bi'])
  bj = jnp.zeros((1, 128), f32).at[0, :C].set(params['bj'])

  edge_pass = _make_edge_pass()
  emb_rows = _make_emb_gather()(params['emb'], vid)
  emb_pad = jnp.pad(emb_rows, ((0, 0), (0, DP - H)))
  h0 = _h0_assemble(emb_pad, selid, selp)

  zinit = jnp.zeros((NC, NP, DP), f32)
  ppart = edge_pass(pos_pad, pidx2, dst2, zinit)

  h = h0
  for _ in range(T):
    tr = _trans(h, w_pad, b_pad)
    apart = edge_pass(tr, gidx2, dst2, ppart)
    h = _gru(apart, h, wi3, wh3, bi3, bh3)

  logits = _readout(h, h0, gb, wi1, wi2, wj, bi, bj)
  return logits[:, :C]


# speedup vs baseline: 3.5736x; 1.0003x over previous
import functools

import numpy as np

import jax
import jax.numpy as jnp
from jax import lax
from jax.experimental import pallas as pl
from jax.experimental.pallas import tpu as pltpu
from jax.experimental.pallas import tpu_sc as plsc

N = 10000
V = 2230
H = 128
SEL = 2
D = H + SEL
ET = 6
PMAX = 4096
C = 2
G = 64
T = 2
E_CTRL, E_DATA, E_CALL = 80000, 64000, 16000
E = 2 * (E_CTRL + E_DATA + E_CALL)

DP = 144
NP = 10112
NC, NS = 2, 16
NW = NC * NS
CHUNK = 64
GG = 18
NG0 = 9
NG1 = 9
CPT0 = NG0 * GG
CPT1 = NG1 * GG
TOTC = NS * (CPT0 + CPT1)
E_PAD = TOTC * CHUNK
ROWS_PT = NP // NS
ECH = 128

RB = 1264
NBLK = NP // RB

_ET_NP = np.concatenate([
    np.full(E_CTRL, 0), np.full(E_DATA, 1), np.full(E_CALL, 2),
    np.full(E_CTRL, 3), np.full(E_DATA, 4), np.full(E_CALL, 5),
]).astype(np.int32)


def _make_edge_pass():
  mesh = plsc.VectorSubcoreMesh(core_axis_name="c", subcore_axis_name="s")

  def body(table, gidx, dstidx, init, out, gbuf, dbuf, rbuf, acc,
           gs0, gs1, gs2, ss0, ss1, ss2, isem):
    c = lax.axis_index("c")
    s = lax.axis_index("s")
    ng = jnp.where(c == 0, NG0, NG1)
    cbase = jnp.where(c == 0, s * CPT0, NS * CPT0 + s * CPT1)
    gsems = [gs0, gs1, gs2]
    ssems = [ss0, ss1, ss2]
    r0 = s * ROWS_PT
    pltpu.sync_copy(init.at[c, pl.ds(r0, ROWS_PT)], acc.at[pl.ds(r0, ROWS_PT)])
    pltpu.sync_copy(gidx.at[pl.ds(cbase, GG)], gbuf.at[0])
    pltpu.sync_copy(dstidx.at[pl.ds(cbase, GG)], dbuf.at[0])
    pltpu.async_copy(table.at[gbuf.at[0].at[0]], rbuf.at[0], gs0)
    pltpu.async_copy(table.at[gbuf.at[0].at[1]], rbuf.at[1], gs1)
    plsc.subcore_barrier()

    def group(g, _):
      gq = lax.rem(g, 2)
      gn = 1 - gq

      @pl.when(g < ng - 1)
      def _():
        pltpu.async_copy(gidx.at[pl.ds(cbase + (g + 1) * GG, GG)],
                         gbuf.at[gn], isem)
        pltpu.async_copy(dstidx.at[pl.ds(cbase + (g + 1) * GG, GG)],
                         dbuf.at[gn], isem)

      def step(jj, _):
        for q in range(3):
          o = jj * 3 + q
          cg = g * GG + o
          nq = (q + 2) % 3
          pltpu.make_async_copy(table.at[gbuf.at[gq].at[o]],
                                rbuf.at[q], gsems[q]).wait()
          pltpu.async_copy(rbuf.at[q], acc.at[dbuf.at[gq].at[o]],
                           ssems[q], add=True)
          @pl.when(jj * 3 + q + 2 < GG if q else cg >= 0)
          def _():
            @pl.when(cg >= 1)
            def _():
              pltpu.make_async_copy(rbuf.at[nq],
                                    acc.at[dbuf.at[gq].at[o]],
                                    ssems[nq]).wait()
            pltpu.async_copy(table.at[gbuf.at[gq].at[o + 2]],
                             rbuf.at[nq], gsems[nq])
        return 0

      lax.fori_loop(0, GG // 3, step, 0)

      @pl.when(g < ng - 1)
      def _():
        pltpu.make_async_copy(gidx.at[pl.ds(cbase + (g + 1) * GG, GG)],
                              gbuf.at[gn], isem).wait()
        pltpu.make_async_copy(dstidx.at[pl.ds(cbase + (g + 1) * GG, GG)],
                              dbuf.at[gn], isem).wait()
        pltpu.make_async_copy(rbuf.at[0], acc.at[dbuf.at[gn].at[0]],
                              ssems[0]).wait()
        pltpu.async_copy(table.at[gbuf.at[gn].at[0]], rbuf.at[0], gs0)
        pltpu.make_async_copy(rbuf.at[1], acc.at[dbuf.at[gn].at[1]],
                              ssems[1]).wait()
        pltpu.async_copy(table.at[gbuf.at[gn].at[1]], rbuf.at[1], gs1)
      return 0

    lax.fori_loop(0, ng, group, 0)
    for q in range(3):
      pltpu.make_async_copy(rbuf.at[q], acc.at[dbuf.at[0].at[0]],
                            ssems[q]).wait()
    plsc.subcore_barrier()
    pltpu.sync_copy(acc.at[pl.ds(r0, ROWS_PT)], out.at[c, pl.ds(r0, ROWS_PT)])

  return pl.kernel(
      body,
      out_type=jax.ShapeDtypeStruct((NC, NP, DP), jnp.float32),
      mesh=mesh,
      scratch_types=[
          pltpu.VMEM((2, GG, CHUNK), jnp.int32),
          pltpu.VMEM((2, GG, CHUNK), jnp.int32),
          pltpu.VMEM((3, CHUNK, DP), jnp.float32),
          pltpu.VMEM_SHARED((NP, DP), jnp.float32),
          pltpu.SemaphoreType.DMA,
          pltpu.SemaphoreType.DMA,
          pltpu.SemaphoreType.DMA,
          pltpu.SemaphoreType.DMA,
          pltpu.SemaphoreType.DMA,
          pltpu.SemaphoreType.DMA,
          pltpu.SemaphoreType.DMA,
      ],
      compiler_params=pltpu.CompilerParams(use_tc_tiling_on_sc=False),
  )


def _make_emb_gather():
  mesh = plsc.VectorSubcoreMesh(core_axis_name="c", subcore_axis_name="s")
  nchunk = NP // ECH

  def body(emb, vid, out, ibuf, rbuf, sem):
    c = lax.axis_index("c")
    s = lax.axis_index("s")
    wid = s * NC + c

    def step(k, _):
      j = wid + k * NW

      @pl.when(j < nchunk)
      def _():
        pltpu.sync_copy(vid.at[j], ibuf)
        pltpu.async_copy(emb.at[ibuf.at[0]], rbuf, sem).wait()
        pltpu.sync_copy(rbuf, out.at[pl.ds(j * ECH, ECH)])
      return 0

    lax.fori_loop(0, (nchunk + NW - 1) // NW, step, 0)

  return pl.kernel(
      body,
      out_type=jax.ShapeDtypeStruct((NP, H), jnp.float32),
      mesh=mesh,
      scratch_types=[
          pltpu.VMEM((1, ECH), jnp.int32),
          pltpu.VMEM((ECH, H), jnp.float32),
          pltpu.SemaphoreType.DMA,
      ],
      compiler_params=pltpu.CompilerParams(use_tc_tiling_on_sc=False),
  )


def _h0_body(embp_ref, selid_ref, selp_ref, out_ref):
  mask = selid_ref[...] == 0
  row = jnp.where(mask, selp_ref[0:1, :], selp_ref[1:2, :])
  out_ref[...] = embp_ref[...] + row


def _h0_assemble(emb_pad, selid, selp):
  return pl.pallas_call(
      _h0_body,
      grid=(NBLK,),
      in_specs=[
          pl.BlockSpec((RB, DP), lambda i: (i, 0)),
          pl.BlockSpec((RB, 1), lambda i: (i, 0)),
          pl.BlockSpec((2, DP), lambda i: (0, 0)),
      ],
      out_specs=pl.BlockSpec((RB, DP), lambda i: (i, 0)),
      out_shape=jax.ShapeDtypeStruct((NP, DP), jnp.float32),
  )(emb_pad, selid, selp)


def _trans_body(h_ref, w_ref, b_ref, out_ref):
  out_ref[...] = (jnp.dot(h_ref[...], w_ref[0],
                          preferred_element_type=jnp.float32)
                  + b_ref[0])


def _trans(h, w_pad, b_pad):
  b_pad = b_pad.reshape(ET, 1, DP)
  return pl.pallas_call(
      _trans_body,
      grid=(ET, NBLK),
      in_specs=[
          pl.BlockSpec((RB, DP), lambda t, i: (i, 0)),
          pl.BlockSpec((1, DP, DP), lambda t, i: (t, 0, 0)),
          pl.BlockSpec((1, 1, DP), lambda t, i: (t, 0, 0)),
      ],
      out_specs=pl.BlockSpec((RB, DP), lambda t, i: (t * NBLK + i, 0)),
      out_shape=jax.ShapeDtypeStruct((ET * NP, DP), jnp.float32),
  )(h, w_pad, b_pad)


def _gru_body(ap_ref, h_ref, wir_ref, wiz_ref, win_ref, whr_ref, whz_ref,
              whn_ref, bir_ref, biz_ref, bin_ref, bhr_ref, bhz_ref, bhn_ref,
              out_ref):
  agg = ap_ref[0] + ap_ref[1]
  h = h_ref[...]
  dot = lambda a, b: jnp.dot(a, b, preferred_element_type=jnp.float32)
  r = jax.nn.sigmoid(dot(agg, wir_ref[...]) + bir_ref[...]
                     + dot(h, whr_ref[...]) + bhr_ref[...])
  z = jax.nn.sigmoid(dot(agg, wiz_ref[...]) + biz_ref[...]
                     + dot(h, whz_ref[...]) + bhz_ref[...])
  n = jnp.tanh(dot(agg, win_ref[...]) + bin_ref[...]
               + r * (dot(h, whn_ref[...]) + bhn_ref[...]))
  out_ref[...] = (1.0 - z) * n + z * h


def _gru(apart, h, wi3, wh3, bi3, bh3):
  mat = lambda: pl.BlockSpec((DP, DP), lambda i: (0, 0))
  vec = lambda: pl.BlockSpec((1, DP), lambda i: (0, 0))
  return pl.pallas_call(
      _gru_body,
      grid=(NBLK,),
      in_specs=[
          pl.BlockSpec((2, RB, DP), lambda i: (0, i, 0)),
          pl.BlockSpec((RB, DP), lambda i: (i, 0)),
          mat(), mat(), mat(), mat(), mat(), mat(),
          vec(), vec(), vec(), vec(), vec(), vec(),
      ],
      out_specs=pl.BlockSpec((RB, DP), lambda i: (i, 0)),
      out_shape=jax.ShapeDtypeStruct((NP, DP), jnp.float32),
  )(apart, h, wi3[0], wi3[1], wi3[2], wh3[0], wh3[1], wh3[2],
    bi3[0], bi3[1], bi3[2], bh3[0], bh3[1], bh3[2])


def _readout_body(h_ref, h0_ref, gb_ref, wi1_ref, wi2_ref, wj_ref, bi_ref,
                  bj_ref, out_ref):
  i = pl.program_id(0)
  dot = lambda a, b: jnp.dot(a, b, preferred_element_type=jnp.float32)
  gate = jax.nn.sigmoid(dot(h_ref[...], wi1_ref[...])
                        + dot(h0_ref[...], wi2_ref[...]) + bi_ref[...])
  val = dot(h_ref[...], wj_ref[...]) + bj_ref[...]
  gv = gate * val
  gidx = jax.lax.broadcasted_iota(jnp.int32, (RB, G), 1)
  oh = (gb_ref[...] == gidx).astype(jnp.float32)
  part = lax.dot_general(oh, gv, (((0,), (0,)), ((), ())),
                         preferred_element_type=jnp.float32)

  @pl.when(i == 0)
  def _():
    out_ref[...] = part

  @pl.when(i > 0)
  def _():
    out_ref[...] = out_ref[...] + part


def _readout(h, h0, gb, wi1, wi2, wj, bi, bj):
  return pl.pallas_call(
      _readout_body,
      grid=(NBLK,),
      in_specs=[
          pl.BlockSpec((RB, DP), lambda i: (i, 0)),
          pl.BlockSpec((RB, DP), lambda i: (i, 0)),
          pl.BlockSpec((RB, 1), lambda i: (i, 0)),
          pl.BlockSpec((DP, 128), lambda i: (0, 0)),
          pl.BlockSpec((DP, 128), lambda i: (0, 0)),
          pl.BlockSpec((DP, 128), lambda i: (0, 0)),
          pl.BlockSpec((1, 128), lambda i: (0, 0)),
          pl.BlockSpec((1, 128), lambda i: (0, 0)),
      ],
      out_specs=pl.BlockSpec((G, 128), lambda i: (0, 0)),
      out_shape=jax.ShapeDtypeStruct((G, 128), jnp.float32),
  )(h, h0, gb, wi1, wi2, wj, bi, bj)


def kernel(node_vocab_ids, node_selector_ids, control_edge_index,
           data_edge_index, call_edge_index, control_pos, data_pos,
           call_pos, graph_batch, params):
  f32 = jnp.float32
  i32 = jnp.int32

  src_f = jnp.concatenate([control_edge_index[0], data_edge_index[0],
                           call_edge_index[0]])
  dst_f = jnp.concatenate([control_edge_index[1], data_edge_index[1],
                           call_edge_index[1]])
  pos_f = jnp.concatenate([control_pos, data_pos, call_pos])
  src = jnp.concatenate([src_f, dst_f])
  dst = jnp.concatenate([dst_f, src_f])
  posa = jnp.concatenate([pos_f, pos_f])
  et = jnp.asarray(_ET_NP)

  gidx = src + et * NP
  pidx = posa
  npad = E_PAD - E
  padr = np.arange(npad, dtype=np.int32)
  pad_gidx = jnp.asarray(padr % (ET * PMAX))
  pad_pidx = jnp.asarray(padr % PMAX)
  pad_dst = jnp.asarray(N + padr % (NP - N))
  gidx2 = jnp.concatenate([gidx, pad_gidx]).reshape(TOTC, CHUNK)
  pidx2 = jnp.concatenate([pidx, pad_pidx]).reshape(TOTC, CHUNK)
  dst2 = jnp.concatenate([dst, pad_dst]).reshape(TOTC, CHUNK)

  vid = jnp.concatenate([node_vocab_ids,
                         jnp.zeros((NP - N,), i32)]).reshape(-1, 1, ECH)
  selid = jnp.concatenate([node_selector_ids,
                           jnp.zeros((NP - N,), i32)]).reshape(NP, 1)
  gb = jnp.concatenate([graph_batch,
                        jnp.full((NP - N,), G, i32)]).reshape(NP, 1)

  selp = jnp.zeros((2, DP), f32).at[:, H:H + SEL].set(params['sel'])
  pos_pad = jnp.pad(params['pos'], ((0, 0), (0, DP - D)))
  b_pad = jnp.pad(params['b'], ((0, 0), (0, DP - D)))
  w_pad = jnp.pad(params['W'], ((0, 0), (0, DP - D), (0, DP - D)))
  wih = params['Wih']
  whh = params['Whh']
  bih = params['bih']
  bhh = params['bhh']
  padm = lambda m: jnp.pad(m, ((0, DP - D), (0, DP - D)))
  padv = lambda v: jnp.pad(v, (0, DP - D)).reshape(1, DP)
  wi3 = [padm(wih[:, k * D:(k + 1) * D]) for k in range(3)]
  wh3 = [padm(whh[:, k * D:(k + 1) * D]) for k in range(3)]
  bi3 = [padv(bih[k * D:(k + 1) * D]) for k in range(3)]
  bh3 = [padv(bhh[k * D:(k + 1) * D]) for k in range(3)]
  wi1 = jnp.zeros((DP, 128), f32).at[:D, :C].set(params['Wi'][:D])
  wi2 = jnp.zeros((DP, 128), f32).at[:D, :C].set(params['Wi'][D:])
  wj = jnp.zeros((DP, 128), f32).at[:D, :C].set(params['Wj'])
  bi = jnp.zeros((1, 128), f32).at[0, :C].set(params['bi'])
  bj = jnp.zeros((1, 128), f32).at[0, :C].set(params['bj'])

  edge_pass = _make_edge_pass()
  emb_rows = _make_emb_gather()(params['emb'], vid)
  emb_pad = jnp.pad(emb_rows, ((0, 0), (0, DP - H)))
  h0 = _h0_assemble(emb_pad, selid, selp)

  zinit = jnp.zeros((NC, NP, DP), f32)
  ppart = edge_pass(pos_pad, pidx2, dst2, zinit)

  h = h0
  for _ in range(T):
    tr = _trans(h, w_pad, b_pad)
    apart = edge_pass(tr, gidx2, dst2, ppart)
    h = _gru(apart, h, wi3, wh3, bi3, bh3)

  logits = _readout(h, h0, gb, wi1, wi2, wj, bi, bj)
  return logits[:, :C]
